# Initial kernel scaffold; baseline (speedup 1.0000x reference)
import functools

import jax
import jax.numpy as jnp
from jax import lax
from jax.experimental import pallas as pl
from jax.experimental.pallas import tpu as pltpu
from jax.experimental.pallas import tpu_sc as plsc

F = 128
NC = 2
NS = 16
NW = NC * NS
L = 16
CH = 512
SB = 128
NSB = CH // SB


def _tc_prep_body(x_ref, w_ref, asd_ref, a3_ref, ea_ref,
                  xt_ref, al_ref, attr_ref, stats_ref):
    xt = jnp.dot(x_ref[...], w_ref[...], preferred_element_type=jnp.float32)
    xt_ref[...] = xt
    al = jnp.dot(xt, asd_ref[...], preferred_element_type=jnp.float32)
    al_ref[...] = al
    sc = jnp.sum(ea_ref[...] * a3_ref[...], axis=1, keepdims=True)
    attr_ref[...] = sc
    m = jnp.maximum(jnp.max(al[:, 0]) + jnp.max(al[:, 1]) + jnp.max(sc), 0.0)
    stats_ref[...] = jnp.full((8, 128), m, jnp.float32)


def _tc_add_body(a_ref, b_ref, o_ref):
    o_ref[...] = a_ref[...] + b_ref[...]


def _seg_kernel_body(np_pad, ept, nch,
                     src_hbm, dst_hbm, attr_hbm, as_hbm, ad_hbm, stats_hbm,
                     eexp_hbm, segpart_hbm,
                     as_v, ad_v, seg_v, src_v, dst_v, attr_v, eexp_v,
                     stat_v, red_v, tmp_v, shared_part):
    cid = lax.axis_index("c")
    sid = lax.axis_index("s")
    tid = cid * NS + sid
    pltpu.sync_copy(as_hbm, as_v)
    pltpu.sync_copy(ad_hbm, ad_v)
    pltpu.sync_copy(stats_hbm.at[0], stat_v)
    m_vec = stat_v[pl.ds(0, L)]
    zero = jnp.zeros((L,), jnp.float32)

    def _zero_body(i, _):
        seg_v[pl.ds(i * L, L)] = zero
        return 0
    lax.fori_loop(0, np_pad // L, _zero_body, 0)

    base = tid * ept

    def _chunk_body(c, _):
        off = base + c * CH
        pltpu.sync_copy(src_hbm.at[pl.ds(off, CH)], src_v)
        pltpu.sync_copy(dst_hbm.at[pl.ds(off, CH)], dst_v)
        pltpu.sync_copy(attr_hbm.at[pl.ds(off, CH)], attr_v)

        def _grp(j, _):
            sl = pl.ds(j * L, L)
            si = src_v[sl]
            di = dst_v[sl]
            a_s = plsc.load_gather(as_v, [si])
            a_d = plsc.load_gather(ad_v, [di])
            lg = a_s + a_d + attr_v[sl]
            lg = jnp.where(lg > 0.0, lg, 0.2 * lg)
            ee = jnp.exp(lg - m_vec)
            eexp_v[sl] = ee
            plsc.addupdate_scatter(seg_v, [di], ee)
            return 0
        lax.fori_loop(0, CH // L, _grp, 0)
        pltpu.sync_copy(eexp_v, eexp_hbm.at[pl.ds(off, CH)])
        return 0
    lax.fori_loop(0, nch, _chunk_body, 0)

    pltpu.sync_copy(seg_v, shared_part.at[sid])
    plsc.subcore_barrier()
    seg_slice = np_pad // NS
    col0 = sid * seg_slice
    pltpu.sync_copy(shared_part.at[0, pl.ds(col0, seg_slice)], red_v)

    def _row_red(r, _):
        pltpu.sync_copy(shared_part.at[r, pl.ds(col0, seg_slice)], tmp_v)

        def _vadd(j, _):
            sl = pl.ds(j * L, L)
            red_v[sl] = red_v[sl] + tmp_v[sl]
            return 0
        lax.fori_loop(0, seg_slice // L, _vadd, 0)
        return 0
    lax.fori_loop(1, NS, _row_red, 0)
    pltpu.sync_copy(red_v, segpart_hbm.at[cid, pl.ds(col0, seg_slice)])


def _agg_kernel_body(np_pad, ept, nch,
                     src_hbm, dst_hbm, ee_hbm, segpart_hbm, xt_hbm,
                     outpart_hbm,
                     seg_v, tmp_v, src_v, dst_v, ee_v, att_v, rows_v,
                     shared_out, sem):
    cid = lax.axis_index("c")
    sid = lax.axis_index("s")
    tid = cid * NS + sid
    pltpu.sync_copy(segpart_hbm.at[0], seg_v)
    pltpu.sync_copy(segpart_hbm.at[1], tmp_v)

    def _vadd(j, _):
        sl = pl.ds(j * L, L)
        seg_v[sl] = seg_v[sl] + tmp_v[sl]
        return 0
    lax.fori_loop(0, np_pad // L, _vadd, 0)

    zero = jnp.zeros((L,), jnp.float32)

    def _zrow(i, _):
        def _zcol(k, _):
            rows_v[i, pl.ds(k * L, L)] = zero
            return 0
        lax.fori_loop(0, F // L, _zcol, 0)
        return 0
    lax.fori_loop(0, CH, _zrow, 0)
    rows_slice = np_pad // NS
    row0 = sid * rows_slice
    pltpu.sync_copy(rows_v, shared_out.at[pl.ds(row0, CH)])
    pltpu.sync_copy(rows_v.at[pl.ds(0, rows_slice - CH)],
                    shared_out.at[pl.ds(row0 + CH, rows_slice - CH)])
    plsc.subcore_barrier()

    base_r = tid * (ept // SB)

    def _chunk_body(c, _):
        r0 = base_r + c * NSB
        pltpu.sync_copy(src_hbm.at[pl.ds(r0, NSB)], src_v)
        pltpu.sync_copy(dst_hbm.at[pl.ds(r0, NSB)], dst_v)
        pltpu.sync_copy(ee_hbm.at[pl.ds(r0, NSB)], ee_v)
        copies = []
        for j in range(NSB):
            copies.append(pltpu.async_copy(
                xt_hbm.at[src_v.at[j]], rows_v.at[pl.ds(j * SB, SB)], sem))
        for d in copies:
            d.wait()

        def _att_grp(j, _):
            def _att_sub(k, _):
                sl = pl.ds(k * L, L)
                di = dst_v[j, sl]
                s = plsc.load_gather(seg_v, [di])
                att_v[j, sl] = ee_v[j, sl] / s
                return 0
            lax.fori_loop(0, SB // L, _att_sub, 0)
            return 0
        lax.fori_loop(0, NSB, _att_grp, 0)

        def _scale_row(i, _):
            a = att_v[i // SB, i % SB]
            for k in range(F // L):
                sl = pl.ds(k * L, L)
                rows_v[i, sl] = rows_v[i, sl] * a
            return 0
        lax.fori_loop(0, CH, _scale_row, 0)

        for j in range(NSB):
            pltpu.sync_copy(rows_v.at[pl.ds(j * SB, SB)],
                            shared_out.at[dst_v.at[j]], add=True)
        return 0
    lax.fori_loop(0, nch, _chunk_body, 0)

    plsc.subcore_barrier()
    pltpu.sync_copy(shared_out.at[pl.ds(row0, rows_slice)],
                    outpart_hbm.at[cid, pl.ds(row0, rows_slice)])


def kernel(x, edge_index, edge_attr, batch, W, a):
    n = x.shape[0]
    e = edge_attr.shape[0]
    ed = edge_attr.shape[1]
    e3 = 2 * e + n
    np_pad = ((n + NS * L * 8 - 1) // (NS * L * 8)) * (NS * L * 8)
    ept = ((e3 + NW * CH - 1) // (NW * CH)) * CH
    nch = ept // CH
    e3p = ept * NW

    src = edge_index[0].astype(jnp.int32)
    dst = edge_index[1].astype(jnp.int32)
    loop = jnp.arange(n, dtype=jnp.int32)
    padi = jnp.full((e3p - e3,), n, dtype=jnp.int32)
    src3 = jnp.concatenate([src, dst, loop, padi])
    dst3 = jnp.concatenate([dst, src, loop, padi])

    x_pad = jnp.pad(x, ((0, np_pad - n), (0, 0)))
    a_sd = a[:2 * F, 0].reshape(2, F).T
    a3_row = a[2 * F:, 0].reshape(1, ed)

    xt_pad, alpha, attr_sc, stats = pl.pallas_call(
        _tc_prep_body,
        out_shape=(
            jax.ShapeDtypeStruct((np_pad, F), jnp.float32),
            jax.ShapeDtypeStruct((np_pad, 2), jnp.float32),
            jax.ShapeDtypeStruct((e, 1), jnp.float32),
            jax.ShapeDtypeStruct((8, 128), jnp.float32),
        ),
    )(x_pad, W, a_sd, a3_row, edge_attr)

    attr3 = jnp.concatenate(
        [attr_sc[:, 0], attr_sc[:, 0], jnp.zeros((e3p - 2 * e,), jnp.float32)])
    alpha_s = alpha[:, 0]
    alpha_d = alpha[:, 1]

    seg_slice = np_pad // NS
    seg_kernel = pl.kernel(
        functools.partial(_seg_kernel_body, np_pad, ept, nch),
        out_type=(
            jax.ShapeDtypeStruct((e3p,), jnp.float32),
            jax.ShapeDtypeStruct((NC, np_pad), jnp.float32),
        ),
        mesh=plsc.VectorSubcoreMesh(core_axis_name="c", subcore_axis_name="s"),
        scratch_types=(
            pltpu.VMEM((np_pad,), jnp.float32),
            pltpu.VMEM((np_pad,), jnp.float32),
            pltpu.VMEM((np_pad,), jnp.float32),
            pltpu.VMEM((CH,), jnp.int32),
            pltpu.VMEM((CH,), jnp.int32),
            pltpu.VMEM((CH,), jnp.float32),
            pltpu.VMEM((CH,), jnp.float32),
            pltpu.VMEM((128,), jnp.float32),
            pltpu.VMEM((seg_slice,), jnp.float32),
            pltpu.VMEM((seg_slice,), jnp.float32),
            pltpu.VMEM_SHARED((NS, np_pad), jnp.float32),
        ),
    )
    e_exp, segpart = seg_kernel(src3, dst3, attr3, alpha_s, alpha_d, stats)

    src2d = src3.reshape(e3p // SB, SB)
    dst2d = dst3.reshape(e3p // SB, SB)
    ee2d = e_exp.reshape(e3p // SB, SB)

    agg_kernel = pl.kernel(
        functools.partial(_agg_kernel_body, np_pad, ept, nch),
        out_type=jax.ShapeDtypeStruct((NC, np_pad, F), jnp.float32),
        mesh=plsc.VectorSubcoreMesh(core_axis_name="c", subcore_axis_name="s"),
        scratch_types=(
            pltpu.VMEM((np_pad,), jnp.float32),
            pltpu.VMEM((np_pad,), jnp.float32),
            pltpu.VMEM((NSB, SB), jnp.int32),
            pltpu.VMEM((NSB, SB), jnp.int32),
            pltpu.VMEM((NSB, SB), jnp.float32),
            pltpu.VMEM((NSB, SB), jnp.float32),
            pltpu.VMEM((CH, F), jnp.float32),
            pltpu.VMEM_SHARED((np_pad, F), jnp.float32),
            pltpu.SemaphoreType.DMA,
        ),
    )
    outpart = agg_kernel(src2d, dst2d, ee2d, segpart, xt_pad)

    out = pl.pallas_call(
        _tc_add_body,
        out_shape=jax.ShapeDtypeStruct((n, F), jnp.float32),
    )(outpart[0, :n, :], outpart[1, :n, :])
    return out


# speedup vs baseline: 13.2710x; 13.2710x over previous
import functools

import jax
import jax.numpy as jnp
from jax import lax
from jax.experimental import pallas as pl
from jax.experimental.pallas import tpu as pltpu
from jax.experimental.pallas import tpu_sc as plsc

F = 128
NC = 2
NS = 16
NW = NC * NS
L = 16
CH = 512
SB = 128
CH2 = 128
NSB = CH2 // SB


def _tc_prep_body(x_ref, w_ref, asd_ref, xt_ref, al_ref, stats_ref):
    xt = jnp.dot(x_ref[...], w_ref[...], preferred_element_type=jnp.float32)
    xt_ref[...] = xt
    al = jnp.dot(xt, asd_ref[...], preferred_element_type=jnp.float32)
    al_ref[...] = al
    m = jnp.maximum(jnp.max(al[:, 0]) + jnp.max(al[:, 1]), 0.0)
    stats_ref[...] = jnp.full((8, 128), m, jnp.float32)


def _tc_attr_body(ea_ref, b_ref, attr_ref):
    attr_ref[...] = jnp.dot(ea_ref[...], b_ref[...],
                            preferred_element_type=jnp.float32)


def _tc_add_body(a_ref, b_ref, o_ref):
    o_ref[...] = a_ref[...] + b_ref[...]


def _seg_kernel_body(np_pad, ept, nch,
                     src_hbm, dst_hbm, attr_hbm, as_hbm, ad_hbm, stats_hbm,
                     eexp_hbm, segpart_hbm,
                     as_v, ad_v, seg_v, src_v, dst_v, attr_v, eexp_v,
                     stat_v, red_v, tmp_v, shared_part):
    cid = lax.axis_index("c")
    sid = lax.axis_index("s")
    tid = cid * NS + sid
    pltpu.sync_copy(as_hbm, as_v)
    pltpu.sync_copy(ad_hbm, ad_v)
    pltpu.sync_copy(stats_hbm.at[0], stat_v)
    m_vec = stat_v[pl.ds(0, L)]
    zero = jnp.zeros((L,), jnp.float32)

    def _zero_body(i, _):
        seg_v[pl.ds(i * L, L)] = zero
        return 0
    lax.fori_loop(0, np_pad // L, _zero_body, 0)

    base = tid * ept

    def _chunk_body(c, _):
        off = base + c * CH
        pltpu.sync_copy(src_hbm.at[pl.ds(off, CH)], src_v)
        pltpu.sync_copy(dst_hbm.at[pl.ds(off, CH)], dst_v)
        pltpu.sync_copy(attr_hbm.at[pl.ds(off, CH)], attr_v)

        def _grp(j, _):
            sl = pl.ds(j * L, L)
            si = src_v[sl]
            di = dst_v[sl]
            a_s = plsc.load_gather(as_v, [si])
            a_d = plsc.load_gather(ad_v, [di])
            lg = a_s + a_d + attr_v[sl]
            lg = jnp.where(lg > 0.0, lg, 0.2 * lg)
            ee = jnp.exp(lg - m_vec)
            eexp_v[sl] = ee
            plsc.addupdate_scatter(seg_v, [di], ee)
            return 0
        lax.fori_loop(0, CH // L, _grp, 0)
        pltpu.sync_copy(eexp_v, eexp_hbm.at[pl.ds(off, CH)])
        return 0
    lax.fori_loop(0, nch, _chunk_body, 0)

    pltpu.sync_copy(seg_v, shared_part.at[sid])
    plsc.subcore_barrier()
    seg_slice = np_pad // NS
    col0 = sid * seg_slice
    pltpu.sync_copy(shared_part.at[0, pl.ds(col0, seg_slice)], red_v)

    def _row_red(r, _):
        pltpu.sync_copy(shared_part.at[r, pl.ds(col0, seg_slice)], tmp_v)

        def _vadd(j, _):
            sl = pl.ds(j * L, L)
            red_v[sl] = red_v[sl] + tmp_v[sl]
            return 0
        lax.fori_loop(0, seg_slice // L, _vadd, 0)
        return 0
    lax.fori_loop(1, NS, _row_red, 0)
    pltpu.sync_copy(red_v, segpart_hbm.at[cid, pl.ds(col0, seg_slice)])


def _agg_kernel_body(np_pad, ept, nch,
                     src_hbm, dst_hbm, ee_hbm, segpart_hbm, xt_hbm,
                     outpart_hbm,
                     seg_v, tmp_v, src_v, dst_v, ee_v, att_v, rows_v,
                     shared_out, sem):
    cid = lax.axis_index("c")
    sid = lax.axis_index("s")
    tid = cid * NS + sid
    pltpu.sync_copy(segpart_hbm.at[0], seg_v)
    pltpu.sync_copy(segpart_hbm.at[1], tmp_v)

    def _vadd(j, _):
        sl = pl.ds(j * L, L)
        seg_v[sl] = seg_v[sl] + tmp_v[sl]
        return 0
    lax.fori_loop(0, np_pad // L, _vadd, 0)

    zero = jnp.zeros((L,), jnp.float32)

    def _zrow(i, _):
        def _zcol(k, _):
            rows_v[i, pl.ds(k * L, L)] = zero
            return 0
        lax.fori_loop(0, F // L, _zcol, 0)
        return 0
    lax.fori_loop(0, CH2, _zrow, 0)
    rows_slice = np_pad // NS
    row0 = sid * rows_slice
    for r in range(rows_slice // CH2):
        pltpu.sync_copy(rows_v, shared_out.at[pl.ds(row0 + r * CH2, CH2)])
    plsc.subcore_barrier()

    base_r = tid * (ept // SB)

    def _chunk_body(c, _):
        r0 = base_r + c * NSB
        pltpu.sync_copy(src_hbm.at[pl.ds(r0, NSB)], src_v)
        pltpu.sync_copy(dst_hbm.at[pl.ds(r0, NSB)], dst_v)
        pltpu.sync_copy(ee_hbm.at[pl.ds(r0, NSB)], ee_v)
        copies = []
        for j in range(NSB):
            copies.append(pltpu.async_copy(
                xt_hbm.at[src_v.at[j]], rows_v.at[pl.ds(j * SB, SB)], sem))
        for d in copies:
            d.wait()

        def _att_grp(j, _):
            def _att_sub(k, _):
                sl = pl.ds(k * L, L)
                di = dst_v[j, sl]
                s = plsc.load_gather(seg_v, [di])
                att_v[j, sl] = ee_v[j, sl] / s
                return 0
            lax.fori_loop(0, SB // L, _att_sub, 0)
            return 0
        lax.fori_loop(0, NSB, _att_grp, 0)

        def _scale_grp(g, _):
            av = att_v[g // (SB // L), pl.ds((g % (SB // L)) * L, L)]
            for lane in range(L):
                a = av[lane]
                row = g * L + lane
                for k in range(F // L):
                    sl = pl.ds(k * L, L)
                    rows_v[row, sl] = rows_v[row, sl] * a
            return 0
        lax.fori_loop(0, CH2 // L, _scale_grp, 0)

        for j in range(NSB):
            pltpu.sync_copy(rows_v.at[pl.ds(j * SB, SB)],
                            shared_out.at[dst_v.at[j]], add=True)
        return 0
    lax.fori_loop(0, nch, _chunk_body, 0)

    plsc.subcore_barrier()
    pltpu.sync_copy(shared_out.at[pl.ds(row0, rows_slice)],
                    outpart_hbm.at[cid, pl.ds(row0, rows_slice)])


def kernel(x, edge_index, edge_attr, batch, W, a):
    n = x.shape[0]
    e = edge_attr.shape[0]
    ed = edge_attr.shape[1]
    e3 = 2 * e + n
    np_pad = ((n + NS * L * 8 - 1) // (NS * L * 8)) * (NS * L * 8)
    ept = ((e3 + NW * CH - 1) // (NW * CH)) * CH
    nch = ept // CH
    e3p = ept * NW

    src = edge_index[0].astype(jnp.int32)
    dst = edge_index[1].astype(jnp.int32)
    loop = jnp.arange(n, dtype=jnp.int32)
    padi = jnp.full((e3p - e3,), n, dtype=jnp.int32)
    src3 = jnp.concatenate([src, dst, loop, padi])
    dst3 = jnp.concatenate([dst, src, loop, padi])

    x_pad = jnp.pad(x, ((0, np_pad - n), (0, 0)))
    a_sd = a[:2 * F, 0].reshape(2, F).T

    xt_pad, alpha, stats = pl.pallas_call(
        _tc_prep_body,
        out_shape=(
            jax.ShapeDtypeStruct((np_pad, F), jnp.float32),
            jax.ShapeDtypeStruct((np_pad, 2), jnp.float32),
            jax.ShapeDtypeStruct((8, 128), jnp.float32),
        ),
    )(x_pad, W, a_sd)

    gp = 128 // ed
    ea128 = edge_attr.reshape(e // gp, 128)
    a3 = a[2 * F:, 0]
    b_blk = jnp.zeros((128, gp), jnp.float32)
    b_blk = b_blk.at[jnp.arange(128), jnp.arange(128) // ed].set(
        jnp.tile(a3, gp))
    attr8 = pl.pallas_call(
        _tc_attr_body,
        out_shape=jax.ShapeDtypeStruct((e // gp, gp), jnp.float32),
    )(ea128, b_blk)
    attr_sc = attr8.reshape(e)

    attr3 = jnp.concatenate(
        [attr_sc, attr_sc, jnp.zeros((e3p - 2 * e,), jnp.float32)])
    alpha_s = alpha[:, 0]
    alpha_d = alpha[:, 1]

    seg_slice = np_pad // NS
    seg_kernel = pl.kernel(
        functools.partial(_seg_kernel_body, np_pad, ept, nch),
        out_type=(
            jax.ShapeDtypeStruct((e3p,), jnp.float32),
            jax.ShapeDtypeStruct((NC, np_pad), jnp.float32),
        ),
        mesh=plsc.VectorSubcoreMesh(core_axis_name="c", subcore_axis_name="s"),
        compiler_params=pltpu.CompilerParams(needs_layout_passes=False),
        scratch_types=(
            pltpu.VMEM((np_pad,), jnp.float32),
            pltpu.VMEM((np_pad,), jnp.float32),
            pltpu.VMEM((np_pad,), jnp.float32),
            pltpu.VMEM((CH,), jnp.int32),
            pltpu.VMEM((CH,), jnp.int32),
            pltpu.VMEM((CH,), jnp.float32),
            pltpu.VMEM((CH,), jnp.float32),
            pltpu.VMEM((128,), jnp.float32),
            pltpu.VMEM((seg_slice,), jnp.float32),
            pltpu.VMEM((seg_slice,), jnp.float32),
            pltpu.VMEM_SHARED((NS, np_pad), jnp.float32),
        ),
    )
    e_exp, segpart = seg_kernel(src3, dst3, attr3, alpha_s, alpha_d, stats)

    src2d = src3.reshape(e3p // SB, SB)
    dst2d = dst3.reshape(e3p // SB, SB)
    ee2d = e_exp.reshape(e3p // SB, SB)

    agg_kernel = pl.kernel(
        functools.partial(_agg_kernel_body, np_pad, ept, ept // CH2),
        out_type=jax.ShapeDtypeStruct((NC, np_pad, F), jnp.float32),
        mesh=plsc.VectorSubcoreMesh(core_axis_name="c", subcore_axis_name="s"),
        compiler_params=pltpu.CompilerParams(needs_layout_passes=False),
        scratch_types=(
            pltpu.VMEM((np_pad,), jnp.float32),
            pltpu.VMEM((np_pad,), jnp.float32),
            pltpu.VMEM((NSB, SB), jnp.int32),
            pltpu.VMEM((NSB, SB), jnp.int32),
            pltpu.VMEM((NSB, SB), jnp.float32),
            pltpu.VMEM((NSB, SB), jnp.float32),
            pltpu.VMEM((CH2, F), jnp.float32),
            pltpu.VMEM_SHARED((np_pad, F), jnp.float32),
            pltpu.SemaphoreType.DMA,
        ),
    )
    outpart = agg_kernel(src2d, dst2d, ee2d, segpart, xt_pad)

    out = pl.pallas_call(
        _tc_add_body,
        out_shape=jax.ShapeDtypeStruct((n, F), jnp.float32),
    )(outpart[0, :n, :], outpart[1, :n, :])
    return out


# speedup vs baseline: 18.2580x; 1.3758x over previous
import functools

import jax
import jax.numpy as jnp
from jax import lax
from jax.experimental import pallas as pl
from jax.experimental.pallas import tpu as pltpu
from jax.experimental.pallas import tpu_sc as plsc

F = 128
NC = 2
NS = 16
NW = NC * NS
L = 16
CH = 512
SB = 128
CH2 = 128
NSB = CH2 // SB


def _tc_prep_body(x_ref, w_ref, asd_ref, xt_ref, al_ref, stats_ref):
    xt = jnp.dot(x_ref[...], w_ref[...], preferred_element_type=jnp.float32)
    xt_ref[...] = xt
    al = jnp.dot(xt, asd_ref[...], preferred_element_type=jnp.float32)
    al_ref[...] = al
    m = jnp.maximum(jnp.max(al[:, 0]) + jnp.max(al[:, 1]), 0.0)
    stats_ref[...] = jnp.full((8, 128), m, jnp.float32)


def _tc_attr_body(ea_ref, b_ref, attr_ref):
    attr_ref[...] = jnp.dot(ea_ref[...], b_ref[...],
                            preferred_element_type=jnp.float32)


def _tc_add_body(a_ref, b_ref, s0_ref, s1_ref, o_ref):
    o_ref[...] = (a_ref[...] + b_ref[...]) / (s0_ref[...] + s1_ref[...])


def _seg_kernel_body(np_pad, ept, nch,
                     src_hbm, dst_hbm, attr_hbm, as_hbm, ad_hbm, stats_hbm,
                     eexp_hbm, segpart_hbm,
                     as_v, ad_v, seg_v, src_v, dst_v, attr_v, eexp_v,
                     stat_v, red_v, tmp_v, shared_part):
    cid = lax.axis_index("c")
    sid = lax.axis_index("s")
    tid = cid * NS + sid
    pltpu.sync_copy(as_hbm, as_v)
    pltpu.sync_copy(ad_hbm, ad_v)
    pltpu.sync_copy(stats_hbm.at[0], stat_v)
    m_vec = stat_v[pl.ds(0, L)]
    zero = jnp.zeros((L,), jnp.float32)

    def _zero_body(i, _):
        seg_v[pl.ds(i * L, L)] = zero
        return 0
    lax.fori_loop(0, np_pad // L, _zero_body, 0)

    base = tid * ept

    def _chunk_body(c, _):
        off = base + c * CH
        pltpu.sync_copy(src_hbm.at[pl.ds(off, CH)], src_v)
        pltpu.sync_copy(dst_hbm.at[pl.ds(off, CH)], dst_v)
        pltpu.sync_copy(attr_hbm.at[pl.ds(off, CH)], attr_v)

        def _grp(j, _):
            sl = pl.ds(j * L, L)
            si = src_v[sl]
            di = dst_v[sl]
            a_s = plsc.load_gather(as_v, [si])
            a_d = plsc.load_gather(ad_v, [di])
            lg = a_s + a_d + attr_v[sl]
            lg = jnp.where(lg > 0.0, lg, 0.2 * lg)
            ee = jnp.exp(lg - m_vec)
            eexp_v[sl] = ee
            plsc.addupdate_scatter(seg_v, [di], ee)
            return 0
        lax.fori_loop(0, CH // L, _grp, 0)
        pltpu.sync_copy(eexp_v, eexp_hbm.at[pl.ds(off, CH)])
        return 0
    lax.fori_loop(0, nch, _chunk_body, 0)

    pltpu.sync_copy(seg_v, shared_part.at[sid])
    plsc.subcore_barrier()
    seg_slice = np_pad // NS
    col0 = sid * seg_slice
    pltpu.sync_copy(shared_part.at[0, pl.ds(col0, seg_slice)], red_v)

    def _row_red(r, _):
        pltpu.sync_copy(shared_part.at[r, pl.ds(col0, seg_slice)], tmp_v)

        def _vadd(j, _):
            sl = pl.ds(j * L, L)
            red_v[sl] = red_v[sl] + tmp_v[sl]
            return 0
        lax.fori_loop(0, seg_slice // L, _vadd, 0)
        return 0
    lax.fori_loop(1, NS, _row_red, 0)
    pltpu.sync_copy(red_v, segpart_hbm.at[cid, pl.ds(col0, seg_slice)])


def _agg_kernel_body(np_pad, ept, nch,
                     src_hbm, dst_hbm, ee_hbm, xt_hbm,
                     outpart_hbm,
                     src_v, dst_v, ee_v, rows_v, shared_out,
                     sem_i0, sem_i1, sem_g0, sem_g1, sem_s0, sem_s1):
    cid = lax.axis_index("c")
    sid = lax.axis_index("s")
    tid = cid * NS + sid
    sem_i = (sem_i0, sem_i1)
    sem_g = (sem_g0, sem_g1)
    sem_s = (sem_s0, sem_s1)
    base_r = tid * (ept // SB)

    def issue_loads(r, b):
        pltpu.async_copy(src_hbm.at[r], src_v.at[b], sem_i[b])
        pltpu.async_copy(dst_hbm.at[r], dst_v.at[b], sem_i[b])
        pltpu.async_copy(ee_hbm.at[r], ee_v.at[b], sem_i[b])

    def wait_loads(b):
        pltpu.make_async_copy(src_hbm.at[0], src_v.at[b], sem_i[b]).wait()
        pltpu.make_async_copy(dst_hbm.at[0], dst_v.at[b], sem_i[b]).wait()
        pltpu.make_async_copy(ee_hbm.at[0], ee_v.at[b], sem_i[b]).wait()

    def issue_gather(b):
        pltpu.async_copy(xt_hbm.at[src_v.at[b]], rows_v.at[b], sem_g[b])

    def wait_gather(b):
        pltpu.make_async_copy(
            xt_hbm.at[src_v.at[b]], rows_v.at[b], sem_g[b]).wait()

    def issue_scatter(b):
        pltpu.async_copy(rows_v.at[b], shared_out.at[dst_v.at[b]],
                         sem_s[b], add=True)

    def wait_scatter(b):
        pltpu.make_async_copy(
            rows_v.at[b], shared_out.at[dst_v.at[b]], sem_s[b]).wait()

    zero = jnp.zeros((L,), jnp.float32)

    def _zrow(i, _):
        def _zcol(k, _):
            rows_v[0, i, pl.ds(k * L, L)] = zero
            return 0
        lax.fori_loop(0, F // L, _zcol, 0)
        return 0
    lax.fori_loop(0, SB, _zrow, 0)
    rows_slice = np_pad // NS
    row0 = sid * rows_slice
    for r in range(rows_slice // SB):
        pltpu.sync_copy(rows_v.at[0], shared_out.at[pl.ds(row0 + r * SB, SB)])
    plsc.subcore_barrier()

    issue_loads(base_r, 0)
    issue_loads(base_r + 1, 1)
    wait_loads(0)
    issue_gather(0)

    def _steady(c2, _):
        for b in (0, 1):
            c = c2 * 2 + b
            nb = 1 - b
            wait_gather(b)

            @pl.when(c >= 1)
            def _():
                wait_scatter(nb)

            @pl.when(c + 1 < nch)
            def _():
                wait_loads(nb)
                issue_gather(nb)

            def _scale_grp(g, _):
                av = ee_v[b, pl.ds(g * L, L)]
                for lane in range(L):
                    aa = av[lane]
                    row = g * L + lane
                    for k in range(F // L):
                        sl = pl.ds(k * L, L)
                        rows_v[b, row, sl] = rows_v[b, row, sl] * aa
                return 0
            lax.fori_loop(0, SB // L, _scale_grp, 0)

            issue_scatter(b)

            @pl.when(c + 2 < nch)
            def _():
                issue_loads(base_r + c + 2, b)
        return 0
    lax.fori_loop(0, nch // 2, _steady, 0)
    wait_scatter((nch - 1) % 2)

    plsc.subcore_barrier()
    pltpu.sync_copy(shared_out.at[pl.ds(row0, rows_slice)],
                    outpart_hbm.at[cid, pl.ds(row0, rows_slice)])


def kernel(x, edge_index, edge_attr, batch, W, a):
    n = x.shape[0]
    e = edge_attr.shape[0]
    ed = edge_attr.shape[1]
    e3 = 2 * e + n
    np_pad = ((n + NS * L * 8 - 1) // (NS * L * 8)) * (NS * L * 8)
    ept = ((e3 + NW * CH - 1) // (NW * CH)) * CH
    nch = ept // CH
    e3p = ept * NW

    src = edge_index[0].astype(jnp.int32)
    dst = edge_index[1].astype(jnp.int32)
    loop = jnp.arange(n, dtype=jnp.int32)
    padi = jnp.full((e3p - e3,), n, dtype=jnp.int32)
    src3 = jnp.concatenate([src, dst, loop, padi])
    dst3 = jnp.concatenate([dst, src, loop, padi])

    x_pad = jnp.pad(x, ((0, np_pad - n), (0, 0)))
    a_sd = a[:2 * F, 0].reshape(2, F).T

    xt_pad, alpha, stats = pl.pallas_call(
        _tc_prep_body,
        out_shape=(
            jax.ShapeDtypeStruct((np_pad, F), jnp.float32),
            jax.ShapeDtypeStruct((np_pad, 2), jnp.float32),
            jax.ShapeDtypeStruct((8, 128), jnp.float32),
        ),
    )(x_pad, W, a_sd)

    gp = 128 // ed
    ea128 = edge_attr.reshape(e // gp, 128)
    a3 = a[2 * F:, 0]
    b_blk = jnp.zeros((128, gp), jnp.float32)
    b_blk = b_blk.at[jnp.arange(128), jnp.arange(128) // ed].set(
        jnp.tile(a3, gp))
    attr8 = pl.pallas_call(
        _tc_attr_body,
        out_shape=jax.ShapeDtypeStruct((e // gp, gp), jnp.float32),
    )(ea128, b_blk)
    attr_sc = attr8.reshape(e)

    attr3 = jnp.concatenate(
        [attr_sc, attr_sc, jnp.zeros((e3p - 2 * e,), jnp.float32)])
    alpha_s = alpha[:, 0]
    alpha_d = alpha[:, 1]

    seg_slice = np_pad // NS
    seg_kernel = pl.kernel(
        functools.partial(_seg_kernel_body, np_pad, ept, nch),
        out_type=(
            jax.ShapeDtypeStruct((e3p,), jnp.float32),
            jax.ShapeDtypeStruct((NC, np_pad), jnp.float32),
        ),
        mesh=plsc.VectorSubcoreMesh(core_axis_name="c", subcore_axis_name="s"),
        compiler_params=pltpu.CompilerParams(needs_layout_passes=False),
        scratch_types=(
            pltpu.VMEM((np_pad,), jnp.float32),
            pltpu.VMEM((np_pad,), jnp.float32),
            pltpu.VMEM((np_pad,), jnp.float32),
            pltpu.VMEM((CH,), jnp.int32),
            pltpu.VMEM((CH,), jnp.int32),
            pltpu.VMEM((CH,), jnp.float32),
            pltpu.VMEM((CH,), jnp.float32),
            pltpu.VMEM((128,), jnp.float32),
            pltpu.VMEM((seg_slice,), jnp.float32),
            pltpu.VMEM((seg_slice,), jnp.float32),
            pltpu.VMEM_SHARED((NS, np_pad), jnp.float32),
        ),
    )
    e_exp, segpart = seg_kernel(src3, dst3, attr3, alpha_s, alpha_d, stats)

    src2d = src3.reshape(e3p // SB, SB)
    dst2d = dst3.reshape(e3p // SB, SB)
    ee2d = e_exp.reshape(e3p // SB, SB)

    agg_kernel = pl.kernel(
        functools.partial(_agg_kernel_body, np_pad, ept, ept // SB),
        out_type=jax.ShapeDtypeStruct((NC, np_pad, F), jnp.float32),
        mesh=plsc.VectorSubcoreMesh(core_axis_name="c", subcore_axis_name="s"),
        compiler_params=pltpu.CompilerParams(needs_layout_passes=False),
        scratch_types=(
            pltpu.VMEM((2, SB), jnp.int32),
            pltpu.VMEM((2, SB), jnp.int32),
            pltpu.VMEM((2, SB), jnp.float32),
            pltpu.VMEM((2, SB, F), jnp.float32),
            pltpu.VMEM_SHARED((np_pad, F), jnp.float32),
            pltpu.SemaphoreType.DMA,
            pltpu.SemaphoreType.DMA,
            pltpu.SemaphoreType.DMA,
            pltpu.SemaphoreType.DMA,
            pltpu.SemaphoreType.DMA,
            pltpu.SemaphoreType.DMA,
        ),
    )
    outpart = agg_kernel(src2d, dst2d, ee2d, xt_pad)

    out = pl.pallas_call(
        _tc_add_body,
        out_shape=jax.ShapeDtypeStruct((n, F), jnp.float32),
    )(outpart[0, :n, :], outpart[1, :n, :],
      segpart[0, :n].reshape(n, 1), segpart[1, :n].reshape(n, 1))
    return out


# speedup vs baseline: 25.7343x; 1.4095x over previous
import functools

import jax
import jax.numpy as jnp
from jax import lax
from jax.experimental import pallas as pl
from jax.experimental.pallas import tpu as pltpu
from jax.experimental.pallas import tpu_sc as plsc

F = 128
NC = 2
NS = 16
NW = NC * NS
L = 16
SB = 64


def _tc_prep_body(x_ref, w_ref, asd_ref, xt_ref, al_ref, stats_ref):
    xt = jnp.dot(x_ref[...], w_ref[...], preferred_element_type=jnp.float32)
    xt_ref[...] = xt
    al = jnp.dot(xt, asd_ref[...], preferred_element_type=jnp.float32)
    al_ref[...] = al
    m = jnp.maximum(jnp.max(al[:, 0]) + jnp.max(al[:, 1]), 0.0)
    stats_ref[...] = jnp.full((8, 128), m, jnp.float32)


def _tc_attr_body(ea_ref, b_ref, attr_ref):
    attr_ref[...] = jnp.dot(ea_ref[...], b_ref[...],
                            preferred_element_type=jnp.float32)


def _tc_div_body(a_ref, b_ref, s0_ref, s1_ref, o_ref):
    o_ref[...] = (a_ref[...] + b_ref[...]) / (s0_ref[...] + s1_ref[...])


def _gat_kernel_body(np_pad, ept, nch,
                     src_hbm, dst_hbm, attr_hbm, as_hbm, ad_hbm, stats_hbm,
                     xt_hbm,
                     segpart_hbm, outpart_hbm,
                     as_v, ad_v, stat_v, zbuf, src_v, dst_v, attr_v, ee_v,
                     rows_v, shared_seg, shared_out,
                     sem_i0, sem_i1, sem_g0, sem_g1, sem_s0, sem_s1,
                     sem_e0, sem_e1):
    cid = lax.axis_index("c")
    sid = lax.axis_index("s")
    tid = cid * NS + sid
    sem_i = (sem_i0, sem_i1)
    sem_g = (sem_g0, sem_g1)
    sem_s = (sem_s0, sem_s1)
    sem_e = (sem_e0, sem_e1)
    base_r = tid * (ept // SB)

    def issue_loads(r, b):
        pltpu.async_copy(src_hbm.at[r], src_v.at[b], sem_i[b])
        pltpu.async_copy(dst_hbm.at[r], dst_v.at[b], sem_i[b])
        pltpu.async_copy(attr_hbm.at[r], attr_v.at[b], sem_i[b])

    def wait_loads(b):
        pltpu.make_async_copy(src_hbm.at[0], src_v.at[b], sem_i[b]).wait()
        pltpu.make_async_copy(dst_hbm.at[0], dst_v.at[b], sem_i[b]).wait()
        pltpu.make_async_copy(attr_hbm.at[0], attr_v.at[b], sem_i[b]).wait()

    def issue_gather(b):
        pltpu.async_copy(xt_hbm.at[src_v.at[b]], rows_v.at[b], sem_g[b])

    def wait_gather(b):
        pltpu.make_async_copy(
            xt_hbm.at[src_v.at[b]], rows_v.at[b], sem_g[b]).wait()

    def issue_scatter(b):
        pltpu.async_copy(rows_v.at[b], shared_out.at[dst_v.at[b]],
                         sem_s[b], add=True)

    def wait_scatter(b):
        pltpu.make_async_copy(
            rows_v.at[b], shared_out.at[dst_v.at[b]], sem_s[b]).wait()

    def issue_escatter(b):
        pltpu.async_copy(ee_v.at[b], shared_seg.at[dst_v.at[b]],
                         sem_e[b], add=True)

    def wait_escatter(b):
        pltpu.make_async_copy(
            ee_v.at[b], shared_seg.at[dst_v.at[b]], sem_e[b]).wait()

    pltpu.sync_copy(as_hbm, as_v)
    pltpu.sync_copy(ad_hbm, ad_v)
    pltpu.sync_copy(stats_hbm.at[0], stat_v)
    m_vec = stat_v[pl.ds(0, L)]

    zero = jnp.zeros((L,), jnp.float32)
    seg_slice = np_pad // NS

    def _zseg(i, _):
        zbuf[pl.ds(i * L, L)] = zero
        return 0
    lax.fori_loop(0, seg_slice // L, _zseg, 0)
    pltpu.sync_copy(zbuf, shared_seg.at[pl.ds(sid * seg_slice, seg_slice)])

    def _zrow(i, _):
        def _zcol(k, _):
            rows_v[0, i, pl.ds(k * L, L)] = zero
            return 0
        lax.fori_loop(0, F // L, _zcol, 0)
        return 0
    lax.fori_loop(0, SB, _zrow, 0)
    rows_slice = np_pad // NS
    row0 = sid * rows_slice
    for r in range(rows_slice // SB):
        pltpu.sync_copy(rows_v.at[0], shared_out.at[pl.ds(row0 + r * SB, SB)])
    plsc.subcore_barrier()

    issue_loads(base_r, 0)
    issue_loads(base_r + 1, 1)
    wait_loads(0)
    issue_gather(0)

    def _steady(c2, _):
        for b in (0, 1):
            c = c2 * 2 + b
            nb = 1 - b
            wait_gather(b)

            @pl.when(c >= 1)
            def _():
                wait_scatter(nb)

            @pl.when(c + 1 < nch)
            def _():
                wait_loads(nb)
                issue_gather(nb)

            @pl.when(c >= 2)
            def _():
                wait_escatter(b)

            def _grp(g, _):
                sl = pl.ds(g * L, L)
                si = src_v[b, sl]
                di = dst_v[b, sl]
                lg = (plsc.load_gather(as_v, [si])
                      + plsc.load_gather(ad_v, [di])
                      + attr_v[b, sl])
                lg = jnp.where(lg > 0.0, lg, 0.2 * lg)
                ee = jnp.exp(lg - m_vec)
                ee_v[b, sl] = ee
                for lane in range(L):
                    aa = ee[lane]
                    row = g * L + lane
                    for k in range(F // L):
                        ck = pl.ds(k * L, L)
                        rows_v[b, row, ck] = rows_v[b, row, ck] * aa
                return 0
            lax.fori_loop(0, SB // L, _grp, 0)

            issue_escatter(b)
            issue_scatter(b)

            @pl.when(c + 2 < nch)
            def _():
                issue_loads(base_r + c + 2, b)
        return 0
    lax.fori_loop(0, nch // 2, _steady, 0)
    wait_scatter((nch - 1) % 2)
    wait_escatter(0)
    wait_escatter(1)

    plsc.subcore_barrier()
    pltpu.sync_copy(shared_seg.at[pl.ds(sid * seg_slice, seg_slice)],
                    segpart_hbm.at[cid, pl.ds(sid * seg_slice, seg_slice)])
    pltpu.sync_copy(shared_out.at[pl.ds(row0, rows_slice)],
                    outpart_hbm.at[cid, pl.ds(row0, rows_slice)])


def kernel(x, edge_index, edge_attr, batch, W, a):
    n = x.shape[0]
    e = edge_attr.shape[0]
    ed = edge_attr.shape[1]
    e3 = 2 * e + n
    np_pad = ((n + NS * L * 8 - 1) // (NS * L * 8)) * (NS * L * 8)
    ept = ((e3 + NW * SB - 1) // (NW * SB)) * SB
    nch = ept // SB
    e3p = ept * NW

    src = edge_index[0].astype(jnp.int32)
    dst = edge_index[1].astype(jnp.int32)
    loop = jnp.arange(n, dtype=jnp.int32)
    padi = jnp.full((e3p - e3,), n, dtype=jnp.int32)
    src3 = jnp.concatenate([src, dst, loop, padi])
    dst3 = jnp.concatenate([dst, src, loop, padi])

    x_pad = jnp.pad(x, ((0, np_pad - n), (0, 0)))
    a_sd = a[:2 * F, 0].reshape(2, F).T

    xt_pad, alpha, stats = pl.pallas_call(
        _tc_prep_body,
        out_shape=(
            jax.ShapeDtypeStruct((np_pad, F), jnp.float32),
            jax.ShapeDtypeStruct((np_pad, 2), jnp.float32),
            jax.ShapeDtypeStruct((8, 128), jnp.float32),
        ),
    )(x_pad, W, a_sd)

    gp = 128 // ed
    ea128 = edge_attr.reshape(e // gp, 128)
    a3 = a[2 * F:, 0]
    b_blk = jnp.zeros((128, gp), jnp.float32)
    b_blk = b_blk.at[jnp.arange(128), jnp.arange(128) // ed].set(
        jnp.tile(a3, gp))
    attr8 = pl.pallas_call(
        _tc_attr_body,
        out_shape=jax.ShapeDtypeStruct((e // gp, gp), jnp.float32),
    )(ea128, b_blk)
    attr_sc = attr8.reshape(e)

    attr3 = jnp.concatenate(
        [attr_sc, attr_sc, jnp.zeros((e3p - 2 * e,), jnp.float32)])

    src2d = src3.reshape(e3p // SB, SB)
    dst2d = dst3.reshape(e3p // SB, SB)
    attr2d = attr3.reshape(e3p // SB, SB)
    alpha_s = alpha[:, 0]
    alpha_d = alpha[:, 1]

    gat_kernel = pl.kernel(
        functools.partial(_gat_kernel_body, np_pad, ept, nch),
        out_type=(
            jax.ShapeDtypeStruct((NC, np_pad), jnp.float32),
            jax.ShapeDtypeStruct((NC, np_pad, F), jnp.float32),
        ),
        mesh=plsc.VectorSubcoreMesh(core_axis_name="c", subcore_axis_name="s"),
        compiler_params=pltpu.CompilerParams(needs_layout_passes=False),
        scratch_types=(
            pltpu.VMEM((np_pad,), jnp.float32),
            pltpu.VMEM((np_pad,), jnp.float32),
            pltpu.VMEM((128,), jnp.float32),
            pltpu.VMEM((np_pad // NS,), jnp.float32),
            pltpu.VMEM((2, SB), jnp.int32),
            pltpu.VMEM((2, SB), jnp.int32),
            pltpu.VMEM((2, SB), jnp.float32),
            pltpu.VMEM((2, SB), jnp.float32),
            pltpu.VMEM((2, SB, F), jnp.float32),
            pltpu.VMEM_SHARED((np_pad,), jnp.float32),
            pltpu.VMEM_SHARED((np_pad, F), jnp.float32),
            pltpu.SemaphoreType.DMA,
            pltpu.SemaphoreType.DMA,
            pltpu.SemaphoreType.DMA,
            pltpu.SemaphoreType.DMA,
            pltpu.SemaphoreType.DMA,
            pltpu.SemaphoreType.DMA,
            pltpu.SemaphoreType.DMA,
            pltpu.SemaphoreType.DMA,
        ),
    )
    segpart, outpart = gat_kernel(src2d, dst2d, attr2d, alpha_s, alpha_d,
                                  stats, xt_pad)

    out = pl.pallas_call(
        _tc_div_body,
        out_shape=jax.ShapeDtypeStruct((n, F), jnp.float32),
    )(outpart[0, :n, :], outpart[1, :n, :],
      segpart[0, :n].reshape(n, 1), segpart[1, :n].reshape(n, 1))
    return out
